# baseline (speedup 1.0000x reference)
import functools

import jax
import jax.numpy as jnp
from jax import lax
from jax.experimental import pallas as pl
from jax.experimental.pallas import tpu as pltpu
from jax.experimental.pallas import tpu_sc as plsc

K = 16
N = 10000
E = 160000
B = 8
D = 128
H = 4
DH = D // H
FF = 4 * D

NC = 2
NS = 16
NW = NC * NS
NPAD = 10112
RPT = NPAD // NS
CH = 128
EPT = 5120
NCH = EPT // CH
EPAD = NW * EPT
CW = 16
SEG = N // B

NSUB = (RPT + CH - 1) // CH


def _sub(i):
    off = i * CH
    ln = CH if i < NSUB - 1 else RPT - (NSUB - 1) * CH
    return off, ln


def _sc_agg_body(h2d, src_flat, dst_flat, zr_h,
                 agg_out,
                 src_c, dst_c, rows_v, acc, sem):
    c = lax.axis_index("c")
    s = lax.axis_index("s")
    w = s * NC + c

    pltpu.sync_copy(zr_h, rows_v)
    for i in range(NSUB):
        off, ln = _sub(i)
        pltpu.sync_copy(rows_v.at[pl.ds(0, ln)],
                        acc.at[pl.ds(s * RPT + off, ln)])

    def frame_body(t, carry):
        plsc.subcore_barrier()
        base0 = (t * NW + w) * EPT

        def chunk_body(j, carry2):
            base = base0 + j * CH
            pltpu.sync_copy(src_flat.at[pl.ds(base, CH)], src_c)
            pltpu.sync_copy(dst_flat.at[pl.ds(base, CH)], dst_c)
            pltpu.async_copy(h2d.at[src_c], rows_v, sem).wait()
            pltpu.sync_copy(rows_v, acc.at[dst_c], add=True)
            return carry2

        lax.fori_loop(0, NCH, chunk_body, 0)
        plsc.subcore_barrier()
        for i in range(NSUB):
            off, ln = _sub(i)
            pltpu.sync_copy(acc.at[pl.ds(s * RPT + off, ln)],
                            rows_v.at[pl.ds(0, ln)])
            pltpu.sync_copy(rows_v.at[pl.ds(0, ln)],
                            agg_out.at[t, c, pl.ds(s * RPT + off, ln)])
        return carry

    lax.fori_loop(0, K, frame_body, 0)


_sc_agg = functools.partial(
    pl.kernel,
    out_type=jax.ShapeDtypeStruct((K, NC, NPAD, D), jnp.float32),
    mesh=plsc.VectorSubcoreMesh(core_axis_name="c", subcore_axis_name="s"),
    scratch_types=[
        pltpu.VMEM((CH,), jnp.int32),
        pltpu.VMEM((CH,), jnp.int32),
        pltpu.VMEM((CH, D), jnp.float32),
        pltpu.VMEM_SHARED((NPAD, D), jnp.float32),
        pltpu.SemaphoreType.DMA,
    ],
)(_sc_agg_body)


BL = 1264


def _tc_sage_body(h_ref, agg_ref, cnt_ref,
                  wl_ref, wr_ref, b_ref, o_ref):
    h = h_ref[0]
    a = agg_ref[0]
    cn = cnt_ref[0]
    mean = a / jnp.maximum(cn, 1.0)
    out = (jnp.dot(mean, wl_ref[...], preferred_element_type=jnp.float32)
           + jnp.dot(h, wr_ref[...], preferred_element_type=jnp.float32)
           + b_ref[...])
    nrm = jnp.sqrt(jnp.sum(out * out, axis=-1, keepdims=True))
    out = out / jnp.maximum(nrm, 1e-12)
    o_ref[0] = jnp.maximum(out, 0.0)


def _tc_sage(h3, aggd, cntd, wlT, wrT, b2d):
    return pl.pallas_call(
        _tc_sage_body,
        grid=(K, NPAD // BL),
        in_specs=[
            pl.BlockSpec((1, BL, D), lambda t, i: (t, i, 0)),
            pl.BlockSpec((1, BL, D), lambda t, i: (t, i, 0)),
            pl.BlockSpec((1, BL, 1), lambda t, i: (t, i, 0)),
            pl.BlockSpec((D, D), lambda t, i: (0, 0)),
            pl.BlockSpec((D, D), lambda t, i: (0, 0)),
            pl.BlockSpec((1, D), lambda t, i: (0, 0)),
        ],
        out_specs=pl.BlockSpec((1, BL, D), lambda t, i: (t, i, 0)),
        out_shape=jax.ShapeDtypeStruct((K, NPAD, D), jnp.float32),
    )(h3, aggd, cntd, wlT, wrT, b2d)


def _tc_pool_body(h_ref, ball_ref, gsum_ref):
    h = h_ref[0]
    balls = jnp.concatenate(
        [h[b * SEG:b * SEG + 1, :] for b in range(B)], axis=0)
    gsums = jnp.concatenate(
        [jnp.sum(h[b * SEG:(b + 1) * SEG, :], axis=0, keepdims=True)
         for b in range(B)], axis=0)
    ball_ref[...] = balls[None]
    gsum_ref[...] = gsums[None]


def _tc_pool(h3):
    return pl.pallas_call(
        _tc_pool_body,
        grid=(K,),
        in_specs=[pl.BlockSpec((1, NPAD, D), lambda t: (t, 0, 0))],
        out_specs=[pl.BlockSpec((1, B, D), lambda t: (t, 0, 0)),
                   pl.BlockSpec((1, B, D), lambda t: (t, 0, 0))],
        out_shape=[jax.ShapeDtypeStruct((K, B, D), jnp.float32),
                   jax.ShapeDtypeStruct((K, B, D), jnp.float32)],
    )(h3)


def _ln(x, g, b):
    m = jnp.mean(x, axis=-1, keepdims=True)
    v = jnp.mean((x - m) * (x - m), axis=-1, keepdims=True)
    return (x - m) * lax.rsqrt(v + 1e-5) * g + b


def _tc_encoder_body(ball_ref, gsum_ref, posx_ref,
                     alng_ref, alnb_ref, aw1_ref, ab1_ref, aw2_ref, ab2_ref,
                     l0_refs, l1_refs, o_ref):
    ball = jnp.swapaxes(ball_ref[...], 0, 1).reshape(B * K, D)
    gmean = jnp.swapaxes(gsum_ref[...], 0, 1).reshape(B * K, D) * (1.0 / SEG)
    f = jnp.concatenate([ball, gmean], axis=-1)
    f = _ln(f, alng_ref[...], alnb_ref[...])
    f = jnp.maximum(jnp.dot(f, aw1_ref[...], preferred_element_type=jnp.float32)
                    + ab1_ref[...], 0.0)
    f = jnp.dot(f, aw2_ref[...], preferred_element_type=jnp.float32) + ab2_ref[...]
    fs = f + posx_ref[...]

    for refs in (l0_refs, l1_refs):
        (ln1g, ln1b, wqkv, bqkv, wo, bo, ln2g, ln2b, w1, b1, w2, b2) = refs
        hh = _ln(fs, ln1g[...], ln1b[...])
        qkv = jnp.dot(hh, wqkv[...], preferred_element_type=jnp.float32) + bqkv[...]
        q = qkv[:, :D]
        k = qkv[:, D:2 * D]
        v = qkv[:, 2 * D:]
        rows = []
        for b in range(B):
            qb = q[b * K:(b + 1) * K, :]
            kb = k[b * K:(b + 1) * K, :]
            vb = v[b * K:(b + 1) * K, :]
            heads = []
            for hd in range(H):
                qh = qb[:, hd * DH:(hd + 1) * DH]
                kh = kb[:, hd * DH:(hd + 1) * DH]
                vh = vb[:, hd * DH:(hd + 1) * DH]
                att = lax.dot_general(qh, kh, (((1,), (1,)), ((), ())),
                                      preferred_element_type=jnp.float32)
                att = att * (1.0 / (DH ** 0.5))
                mx = jnp.max(att, axis=-1, keepdims=True)
                ex = jnp.exp(att - mx)
                att = ex / jnp.sum(ex, axis=-1, keepdims=True)
                heads.append(jnp.dot(att, vh, preferred_element_type=jnp.float32))
            rows.append(jnp.concatenate(heads, axis=-1))
        o = jnp.concatenate(rows, axis=0)
        fs = fs + jnp.dot(o, wo[...], preferred_element_type=jnp.float32) + bo[...]
        h2 = _ln(fs, ln2g[...], ln2b[...])
        h2 = jnp.maximum(jnp.dot(h2, w1[...], preferred_element_type=jnp.float32)
                         + b1[...], 0.0)
        h2 = jnp.dot(h2, w2[...], preferred_element_type=jnp.float32) + b2[...]
        fs = fs + h2

    o_ref[...] = jnp.concatenate(
        [fs[b * K + K - 1:b * K + K, :] for b in range(B)], axis=0)


def _tc_encoder(ball, gsum, posx, aggp, layerp):
    return pl.pallas_call(
        _tc_encoder_body,
        out_shape=jax.ShapeDtypeStruct((B, D), jnp.float32),
    )(ball, gsum, posx, *aggp, layerp[0], layerp[1])


def kernel(x, edge_index, batch_index, ptr, params):
    f32 = jnp.float32

    xpad = jnp.concatenate([x, jnp.zeros((K, NPAD - N, D), f32)], axis=1)

    src = edge_index[:, 0, :]
    dst = edge_index[:, 1, :]
    pad = EPAD - E
    src_p = jnp.concatenate([src, jnp.zeros((K, pad), jnp.int32)], axis=1)
    dst_p = jnp.concatenate(
        [dst, jnp.full((K, pad), NPAD - 1, jnp.int32)], axis=1)
    frame_off = (jnp.arange(K, dtype=jnp.int32) * NPAD)[:, None]
    src_g = (src_p + frame_off).reshape(K * EPAD)
    dst_g = dst_p.reshape(K * EPAD)

    zr_h = jnp.zeros((CH, D), f32)
    ones_tab = jnp.ones((8, D), f32)
    src_ones = jnp.zeros((K * EPAD,), jnp.int32)

    def _diff(cum):
        ss = cum[:, 0] + cum[:, 1]
        return ss - jnp.concatenate([jnp.zeros_like(ss[:1]), ss[:-1]], axis=0)

    g = params["gnn"]
    cnt = _sc_agg(ones_tab, src_ones, dst_g, zr_h)
    cntd = _diff(cnt[:, :, :, 0:1])
    agg1 = _sc_agg(xpad.reshape(K * NPAD, D), src_g, dst_g, zr_h)
    h1 = _tc_sage(xpad, _diff(agg1), cntd,
                  g[0]["Wl"].T, g[0]["Wr"].T, g[0]["b"][None, :])
    agg2 = _sc_agg(h1.reshape(K * NPAD, D), src_g, dst_g, zr_h)
    h2 = _tc_sage(h1, _diff(agg2), cntd,
                  g[1]["Wl"].T, g[1]["Wr"].T, g[1]["b"][None, :])

    ball, gsum = _tc_pool(h2)

    a = params["agg"]
    aggp = (a["ln_g"][None, :], a["ln_b"][None, :],
            a["W1"].T, a["b1"][None, :], a["W2"].T, a["b2"][None, :])
    layerp = []
    for p in params["layers"]:
        layerp.append((p["ln1_g"][None, :], p["ln1_b"][None, :],
                       p["Wqkv"].T, p["bqkv"][None, :],
                       p["Wo"].T, p["bo"][None, :],
                       p["ln2_g"][None, :], p["ln2_b"][None, :],
                       p["W1"].T, p["b1"][None, :],
                       p["W2"].T, p["b2"][None, :]))
    posx = jnp.tile(params["pos"], (B, 1))

    z = _tc_encoder(ball, gsum, posx, aggp, layerp)
    last_node = h2[K - 1, :N, :]
    return (z, last_node)


# speedup vs baseline: 10.8605x; 10.8605x over previous
import functools

import jax
import jax.numpy as jnp
from jax import lax
from jax.experimental import pallas as pl
from jax.experimental.pallas import tpu as pltpu
from jax.experimental.pallas import tpu_sc as plsc

K = 16
N = 10000
E = 160000
B = 8
D = 128
H = 4
DH = D // H
FF = 4 * D

NC = 2
NS = 16
NW = NC * NS
NPAD = 10112
RPT = NPAD // NS
CH = 128
EPT = 5120
NCH = EPT // CH
EPAD = NW * EPT
CW = 16
SEG = N // B

NSUB = (RPT + CH - 1) // CH


def _sub(i):
    off = i * CH
    ln = CH if i < NSUB - 1 else RPT - (NSUB - 1) * CH
    return off, ln


def _sc_agg_body(h2d, src_flat, dst_flat, zr_h,
                 agg_out,
                 srcA, dstA, srcB, dstB, rowsA, rowsB, acc,
                 si1, si2, sg1, sg2, ss1, ss2):
    c = lax.axis_index("c")
    s = lax.axis_index("s")
    w = s * NC + c

    pltpu.sync_copy(zr_h, rowsA)
    for i in range(NSUB):
        off, ln = _sub(i)
        pltpu.sync_copy(rowsA.at[pl.ds(0, ln)],
                        acc.at[pl.ds(s * RPT + off, ln)])

    def frame_body(t, carry):
        plsc.subcore_barrier()
        base0 = (t * NW + w) * EPT

        def chunk_body(j2, carry2):
            a = base0 + (2 * j2) * CH
            b = a + CH
            ia_s = pltpu.async_copy(src_flat.at[pl.ds(a, CH)], srcA, si1)
            ia_d = pltpu.async_copy(dst_flat.at[pl.ds(a, CH)], dstA, si1)
            ib_s = pltpu.async_copy(src_flat.at[pl.ds(b, CH)], srcB, si2)
            ib_d = pltpu.async_copy(dst_flat.at[pl.ds(b, CH)], dstB, si2)
            ia_s.wait(); ia_d.wait()
            gA = pltpu.async_copy(h2d.at[srcA], rowsA, sg1)
            ib_s.wait(); ib_d.wait()
            gB = pltpu.async_copy(h2d.at[srcB], rowsB, sg2)
            gA.wait()
            sA = pltpu.async_copy(rowsA, acc.at[dstA], ss1, add=True)
            gB.wait()
            sB = pltpu.async_copy(rowsB, acc.at[dstB], ss2, add=True)
            sA.wait(); sB.wait()
            return carry2

        lax.fori_loop(0, NCH // 2, chunk_body, 0)
        plsc.subcore_barrier()
        for i in range(NSUB):
            off, ln = _sub(i)
            pltpu.sync_copy(acc.at[pl.ds(s * RPT + off, ln)],
                            rowsA.at[pl.ds(0, ln)])
            pltpu.sync_copy(rowsA.at[pl.ds(0, ln)],
                            agg_out.at[t, c, pl.ds(s * RPT + off, ln)])
        return carry

    lax.fori_loop(0, K, frame_body, 0)


_sc_agg = functools.partial(
    pl.kernel,
    out_type=jax.ShapeDtypeStruct((K, NC, NPAD, D), jnp.float32),
    mesh=plsc.VectorSubcoreMesh(core_axis_name="c", subcore_axis_name="s"),
    scratch_types=[
        pltpu.VMEM((CH,), jnp.int32),
        pltpu.VMEM((CH,), jnp.int32),
        pltpu.VMEM((CH,), jnp.int32),
        pltpu.VMEM((CH,), jnp.int32),
        pltpu.VMEM((CH, D), jnp.float32),
        pltpu.VMEM((CH, D), jnp.float32),
        pltpu.VMEM_SHARED((NPAD, D), jnp.float32),
        pltpu.SemaphoreType.DMA,
        pltpu.SemaphoreType.DMA,
        pltpu.SemaphoreType.DMA,
        pltpu.SemaphoreType.DMA,
        pltpu.SemaphoreType.DMA,
        pltpu.SemaphoreType.DMA,
    ],
)(_sc_agg_body)


def _sc_cnt_body(dst_flat, ones_h, zr_h,
                 cnt_out,
                 dstA, dstB, onesr, cacc,
                 si1, si2, ss1, ss2):
    c = lax.axis_index("c")
    s = lax.axis_index("s")
    w = s * NC + c

    pltpu.sync_copy(zr_h, onesr)
    for i in range(NSUB):
        off, ln = _sub(i)
        pltpu.sync_copy(onesr.at[pl.ds(0, ln)],
                        cacc.at[pl.ds(s * RPT + off, ln)])

    def frame_body(t, carry):
        pltpu.sync_copy(ones_h, onesr)
        plsc.subcore_barrier()
        base0 = (t * NW + w) * EPT

        def chunk_body(j2, carry2):
            a = base0 + (2 * j2) * CH
            b = a + CH
            iA = pltpu.async_copy(dst_flat.at[pl.ds(a, CH)], dstA, si1)
            iB = pltpu.async_copy(dst_flat.at[pl.ds(b, CH)], dstB, si2)
            iA.wait()
            sA = pltpu.async_copy(onesr, cacc.at[dstA], ss1, add=True)
            iB.wait()
            sB = pltpu.async_copy(onesr, cacc.at[dstB], ss2, add=True)
            sA.wait(); sB.wait()
            return carry2

        lax.fori_loop(0, NCH // 2, chunk_body, 0)
        plsc.subcore_barrier()
        for i in range(NSUB):
            off, ln = _sub(i)
            pltpu.sync_copy(cacc.at[pl.ds(s * RPT + off, ln)],
                            onesr.at[pl.ds(0, ln)])
            pltpu.sync_copy(onesr.at[pl.ds(0, ln)],
                            cnt_out.at[t, c, pl.ds(s * RPT + off, ln)])
        return carry

    lax.fori_loop(0, K, frame_body, 0)


_sc_cnt = functools.partial(
    pl.kernel,
    out_type=jax.ShapeDtypeStruct((K, NC, NPAD, D), jnp.float32),
    mesh=plsc.VectorSubcoreMesh(core_axis_name="c", subcore_axis_name="s"),
    scratch_types=[
        pltpu.VMEM((CH,), jnp.int32),
        pltpu.VMEM((CH,), jnp.int32),
        pltpu.VMEM((CH, D), jnp.float32),
        pltpu.VMEM_SHARED((NPAD, D), jnp.float32),
        pltpu.SemaphoreType.DMA,
        pltpu.SemaphoreType.DMA,
        pltpu.SemaphoreType.DMA,
        pltpu.SemaphoreType.DMA,
    ],
)(_sc_cnt_body)


BL = 1264


def _tc_sage_body(h_ref, agg_ref, cnt_ref,
                  wl_ref, wr_ref, b_ref, o_ref):
    h = h_ref[0]
    a = agg_ref[0]
    cn = cnt_ref[0]
    mean = a / jnp.maximum(cn, 1.0)
    out = (jnp.dot(mean, wl_ref[...], preferred_element_type=jnp.float32)
           + jnp.dot(h, wr_ref[...], preferred_element_type=jnp.float32)
           + b_ref[...])
    nrm = jnp.sqrt(jnp.sum(out * out, axis=-1, keepdims=True))
    out = out / jnp.maximum(nrm, 1e-12)
    o_ref[0] = jnp.maximum(out, 0.0)


def _tc_sage(h3, aggd, cntd, wlT, wrT, b2d):
    return pl.pallas_call(
        _tc_sage_body,
        grid=(K, NPAD // BL),
        in_specs=[
            pl.BlockSpec((1, BL, D), lambda t, i: (t, i, 0)),
            pl.BlockSpec((1, BL, D), lambda t, i: (t, i, 0)),
            pl.BlockSpec((1, BL, 1), lambda t, i: (t, i, 0)),
            pl.BlockSpec((D, D), lambda t, i: (0, 0)),
            pl.BlockSpec((D, D), lambda t, i: (0, 0)),
            pl.BlockSpec((1, D), lambda t, i: (0, 0)),
        ],
        out_specs=pl.BlockSpec((1, BL, D), lambda t, i: (t, i, 0)),
        out_shape=jax.ShapeDtypeStruct((K, NPAD, D), jnp.float32),
    )(h3, aggd, cntd, wlT, wrT, b2d)


def _tc_pool_body(h_ref, ball_ref, gsum_ref):
    h = h_ref[0]
    balls = jnp.concatenate(
        [h[b * SEG:b * SEG + 1, :] for b in range(B)], axis=0)
    gsums = jnp.concatenate(
        [jnp.sum(h[b * SEG:(b + 1) * SEG, :], axis=0, keepdims=True)
         for b in range(B)], axis=0)
    ball_ref[...] = balls[None]
    gsum_ref[...] = gsums[None]


def _tc_pool(h3):
    return pl.pallas_call(
        _tc_pool_body,
        grid=(K,),
        in_specs=[pl.BlockSpec((1, NPAD, D), lambda t: (t, 0, 0))],
        out_specs=[pl.BlockSpec((1, B, D), lambda t: (t, 0, 0)),
                   pl.BlockSpec((1, B, D), lambda t: (t, 0, 0))],
        out_shape=[jax.ShapeDtypeStruct((K, B, D), jnp.float32),
                   jax.ShapeDtypeStruct((K, B, D), jnp.float32)],
    )(h3)


def _ln(x, g, b):
    m = jnp.mean(x, axis=-1, keepdims=True)
    v = jnp.mean((x - m) * (x - m), axis=-1, keepdims=True)
    return (x - m) * lax.rsqrt(v + 1e-5) * g + b


def _tc_encoder_body(ball_ref, gsum_ref, posx_ref,
                     alng_ref, alnb_ref, aw1_ref, ab1_ref, aw2_ref, ab2_ref,
                     l0_refs, l1_refs, o_ref):
    ball = jnp.swapaxes(ball_ref[...], 0, 1).reshape(B * K, D)
    gmean = jnp.swapaxes(gsum_ref[...], 0, 1).reshape(B * K, D) * (1.0 / SEG)
    f = jnp.concatenate([ball, gmean], axis=-1)
    f = _ln(f, alng_ref[...], alnb_ref[...])
    f = jnp.maximum(jnp.dot(f, aw1_ref[...], preferred_element_type=jnp.float32)
                    + ab1_ref[...], 0.0)
    f = jnp.dot(f, aw2_ref[...], preferred_element_type=jnp.float32) + ab2_ref[...]
    fs = f + posx_ref[...]

    for refs in (l0_refs, l1_refs):
        (ln1g, ln1b, wqkv, bqkv, wo, bo, ln2g, ln2b, w1, b1, w2, b2) = refs
        hh = _ln(fs, ln1g[...], ln1b[...])
        qkv = jnp.dot(hh, wqkv[...], preferred_element_type=jnp.float32) + bqkv[...]
        q = qkv[:, :D]
        k = qkv[:, D:2 * D]
        v = qkv[:, 2 * D:]
        rows = []
        for b in range(B):
            qb = q[b * K:(b + 1) * K, :]
            kb = k[b * K:(b + 1) * K, :]
            vb = v[b * K:(b + 1) * K, :]
            heads = []
            for hd in range(H):
                qh = qb[:, hd * DH:(hd + 1) * DH]
                kh = kb[:, hd * DH:(hd + 1) * DH]
                vh = vb[:, hd * DH:(hd + 1) * DH]
                att = lax.dot_general(qh, kh, (((1,), (1,)), ((), ())),
                                      preferred_element_type=jnp.float32)
                att = att * (1.0 / (DH ** 0.5))
                mx = jnp.max(att, axis=-1, keepdims=True)
                ex = jnp.exp(att - mx)
                att = ex / jnp.sum(ex, axis=-1, keepdims=True)
                heads.append(jnp.dot(att, vh, preferred_element_type=jnp.float32))
            rows.append(jnp.concatenate(heads, axis=-1))
        o = jnp.concatenate(rows, axis=0)
        fs = fs + jnp.dot(o, wo[...], preferred_element_type=jnp.float32) + bo[...]
        h2 = _ln(fs, ln2g[...], ln2b[...])
        h2 = jnp.maximum(jnp.dot(h2, w1[...], preferred_element_type=jnp.float32)
                         + b1[...], 0.0)
        h2 = jnp.dot(h2, w2[...], preferred_element_type=jnp.float32) + b2[...]
        fs = fs + h2

    o_ref[...] = jnp.concatenate(
        [fs[b * K + K - 1:b * K + K, :] for b in range(B)], axis=0)


def _tc_encoder(ball, gsum, posx, aggp, layerp):
    return pl.pallas_call(
        _tc_encoder_body,
        out_shape=jax.ShapeDtypeStruct((B, D), jnp.float32),
    )(ball, gsum, posx, *aggp, layerp[0], layerp[1])


def kernel(x, edge_index, batch_index, ptr, params):
    f32 = jnp.float32

    xpad = jnp.concatenate([x, jnp.zeros((K, NPAD - N, D), f32)], axis=1)

    src = edge_index[:, 0, :]
    dst = edge_index[:, 1, :]
    pad = EPAD - E
    src_p = jnp.concatenate([src, jnp.zeros((K, pad), jnp.int32)], axis=1)
    dst_p = jnp.concatenate(
        [dst, jnp.full((K, pad), NPAD - 1, jnp.int32)], axis=1)
    frame_off = (jnp.arange(K, dtype=jnp.int32) * NPAD)[:, None]
    src_g = (src_p + frame_off).reshape(K * EPAD)
    dst_g = dst_p.reshape(K * EPAD)

    zr_h = jnp.zeros((CH, D), f32)
    ones_big = jnp.ones((CH, D), f32)

    def _diff(cum):
        ss = cum[:, 0] + cum[:, 1]
        return ss - jnp.concatenate([jnp.zeros_like(ss[:1]), ss[:-1]], axis=0)

    g = params["gnn"]
    cnt = _sc_cnt(dst_g, ones_big, zr_h)
    cntd = _diff(cnt[:, :, :, 0:1])
    agg1 = _sc_agg(xpad.reshape(K * NPAD, D), src_g, dst_g, zr_h)
    h1 = _tc_sage(xpad, _diff(agg1), cntd,
                  g[0]["Wl"].T, g[0]["Wr"].T, g[0]["b"][None, :])
    agg2 = _sc_agg(h1.reshape(K * NPAD, D), src_g, dst_g, zr_h)
    h2 = _tc_sage(h1, _diff(agg2), cntd,
                  g[1]["Wl"].T, g[1]["Wr"].T, g[1]["b"][None, :])

    ball, gsum = _tc_pool(h2)

    a = params["agg"]
    aggp = (a["ln_g"][None, :], a["ln_b"][None, :],
            a["W1"].T, a["b1"][None, :], a["W2"].T, a["b2"][None, :])
    layerp = []
    for p in params["layers"]:
        layerp.append((p["ln1_g"][None, :], p["ln1_b"][None, :],
                       p["Wqkv"].T, p["bqkv"][None, :],
                       p["Wo"].T, p["bo"][None, :],
                       p["ln2_g"][None, :], p["ln2_b"][None, :],
                       p["W1"].T, p["b1"][None, :],
                       p["W2"].T, p["b2"][None, :]))
    posx = jnp.tile(params["pos"], (B, 1))

    z = _tc_encoder(ball, gsum, posx, aggp, layerp)
    last_node = h2[K - 1, :N, :]
    return (z, last_node)
